# Initial kernel scaffold; baseline (speedup 1.0000x reference)
import jax
import jax.numpy as jnp
from jax.experimental import pallas as pl


def kernel(predictions, targets, station_ids):
    raise NotImplementedError("write your pallas kernel here")


# speedup vs baseline: 68.5046x; 68.5046x over previous
import functools

import jax
import jax.numpy as jnp
from jax import lax
from jax.experimental import pallas as pl
from jax.experimental.pallas import tpu as pltpu
from jax.experimental.pallas import tpu_sc as plsc

NSTATIONS = 50000
S_PAD = 50176
NC = 2
NS = 16
NW = NC * NS
SLICE = S_PAD // NS
C_SUB = 1280


def _sc_accumulate_body(p_hbm, t_hbm, s_hbm, out_hbm,
                        s_v, t_v, p_v, t2_v, r2_v, one_v, z_v,
                        tbl_c, tbl_t, tbl_t2, tbl_r2):
    cid = lax.axis_index("c")
    sid = lax.axis_index("s")
    wid = cid * NS + sid

    def _zfill(i, _):
        z_v[pl.ds(i * 16, 16)] = jnp.zeros((16,), jnp.float32)
        return 0
    lax.fori_loop(0, SLICE // 16, _zfill, 0)

    def _onefill(i, _):
        one_v[pl.ds(i * 16, 16)] = jnp.ones((16,), jnp.float32)
        return 0
    lax.fori_loop(0, C_SUB // 16, _onefill, 0)

    base = sid * SLICE
    pltpu.sync_copy(z_v, tbl_c.at[pl.ds(base, SLICE)])
    pltpu.sync_copy(z_v, tbl_t.at[pl.ds(base, SLICE)])
    pltpu.sync_copy(z_v, tbl_t2.at[pl.ds(base, SLICE)])
    pltpu.sync_copy(z_v, tbl_r2.at[pl.ds(base, SLICE)])
    plsc.subcore_barrier()

    n = p_hbm.shape[0]
    nsub = n // C_SUB
    nsub_w = (nsub - wid + NW - 1) // NW

    def _sub(it, _):
        e0 = (wid + it * NW) * C_SUB
        pltpu.sync_copy(p_hbm.at[pl.ds(e0, C_SUB)], p_v)
        pltpu.sync_copy(t_hbm.at[pl.ds(e0, C_SUB)], t_v)
        pltpu.sync_copy(s_hbm.at[pl.ds(e0, C_SUB)], s_v)
        for i in range(C_SUB // 16):
            sl = pl.ds(i * 16, 16)
            tt = t_v[sl]
            pp = p_v[sl]
            d = tt - pp
            t2_v[sl] = tt * tt
            r2_v[sl] = d * d
        for j in range(C_SUB // 128):
            rs = pl.ds(j * 128, 128)
            idx = s_v.at[rs]
            pltpu.sync_copy(one_v.at[rs], tbl_c.at[idx], add=True)
            pltpu.sync_copy(t_v.at[rs], tbl_t.at[idx], add=True)
            pltpu.sync_copy(t2_v.at[rs], tbl_t2.at[idx], add=True)
            pltpu.sync_copy(r2_v.at[rs], tbl_r2.at[idx], add=True)
        return 0

    lax.fori_loop(0, nsub_w, _sub, 0)
    plsc.subcore_barrier()

    out0 = cid * (4 * S_PAD) + base
    for k, tbl in enumerate((tbl_c, tbl_t, tbl_t2, tbl_r2)):
        pltpu.sync_copy(tbl.at[pl.ds(base, SLICE)], z_v)
        pltpu.sync_copy(z_v, out_hbm.at[pl.ds(out0 + k * S_PAD, SLICE)])


@functools.partial(
    pl.kernel,
    out_type=jax.ShapeDtypeStruct((2 * 4 * S_PAD,), jnp.float32),
    mesh=plsc.VectorSubcoreMesh(core_axis_name="c", subcore_axis_name="s",
                                num_cores=NC, num_subcores=NS),
    scratch_types=[
        pltpu.VMEM((C_SUB,), jnp.int32),
        pltpu.VMEM((C_SUB,), jnp.float32),
        pltpu.VMEM((C_SUB,), jnp.float32),
        pltpu.VMEM((C_SUB,), jnp.float32),
        pltpu.VMEM((C_SUB,), jnp.float32),
        pltpu.VMEM((C_SUB,), jnp.float32),
        pltpu.VMEM((SLICE,), jnp.float32),
        pltpu.VMEM_SHARED((S_PAD,), jnp.float32),
        pltpu.VMEM_SHARED((S_PAD,), jnp.float32),
        pltpu.VMEM_SHARED((S_PAD,), jnp.float32),
        pltpu.VMEM_SHARED((S_PAD,), jnp.float32),
    ],
)
def _sc_accumulate(*args):
    _sc_accumulate_body(*args)


def _finalize_body(x_ref, o_ref):
    c = x_ref[0] + x_ref[4]
    st = x_ref[1] + x_ref[5]
    st2 = x_ref[2] + x_ref[6]
    sr = x_ref[3] + x_ref[7]
    cs = jnp.maximum(c, 1.0)
    ss_tot = st2 - st * st / cs
    mse = sr / cs
    ss_tot_safe = jnp.where(ss_tot > 1e-8, ss_tot, 1.0)
    r2 = 1.0 - sr / ss_tot_safe
    r2 = jnp.clip(r2, -1.0, 1.0)
    loss_r2 = 1.0 - r2
    use_mse = (c < 5.0) | (ss_tot <= 1e-8)
    sl = jnp.where(use_mse, mse, loss_r2)
    nonempty = c > 0.0
    sl = jnp.where(nonempty, sl, 0.0)
    n_uniq = jnp.sum(nonempty.astype(jnp.float32))
    val = jnp.sum(sl) / jnp.maximum(n_uniq, 1.0)
    o_ref[...] = val[None, None]


def kernel(predictions, targets, station_ids):
    s1 = station_ids.astype(jnp.int32)
    partials = _sc_accumulate(predictions, targets, s1)
    x = partials.reshape(8, S_PAD // 128, 128)
    out = pl.pallas_call(
        _finalize_body,
        out_shape=jax.ShapeDtypeStruct((1, 1), jnp.float32),
    )(x)
    return out.reshape(())


# speedup vs baseline: 77.5880x; 1.1326x over previous
import functools

import jax
import jax.numpy as jnp
from jax import lax
from jax.experimental import pallas as pl
from jax.experimental.pallas import tpu as pltpu
from jax.experimental.pallas import tpu_sc as plsc

NSTATIONS = 50000
S_PAD = 50176
NC = 2
NS = 16
NW = NC * NS
SLICE = S_PAD // NS
C_SUB = 1280


def _sc_accumulate_body(p_hbm, t_hbm, s_hbm, out_hbm,
                        s_v, t_v, p_v, t2_v, r2_v, one_v, z_v,
                        tbl_c, tbl_t, tbl_t2, tbl_r2,
                        sem0, sem1, sem2, sem3):
    cid = lax.axis_index("c")
    sid = lax.axis_index("s")
    wid = cid * NS + sid

    def _zfill(i, _):
        z_v[pl.ds(i * 16, 16)] = jnp.zeros((16,), jnp.float32)
        return 0
    lax.fori_loop(0, SLICE // 16, _zfill, 0)

    def _onefill(i, _):
        one_v[pl.ds(i * 16, 16)] = jnp.ones((16,), jnp.float32)
        return 0
    lax.fori_loop(0, C_SUB // 16, _onefill, 0)

    base = sid * SLICE
    pltpu.sync_copy(z_v, tbl_c.at[pl.ds(base, SLICE)])
    pltpu.sync_copy(z_v, tbl_t.at[pl.ds(base, SLICE)])
    pltpu.sync_copy(z_v, tbl_t2.at[pl.ds(base, SLICE)])
    pltpu.sync_copy(z_v, tbl_r2.at[pl.ds(base, SLICE)])
    plsc.subcore_barrier()

    n = p_hbm.shape[0]
    nsub = n // C_SUB
    nsub_w = (nsub - wid + NW - 1) // NW

    def _sub(it, _):
        e0 = (wid + it * NW) * C_SUB
        pltpu.sync_copy(p_hbm.at[pl.ds(e0, C_SUB)], p_v)
        pltpu.sync_copy(t_hbm.at[pl.ds(e0, C_SUB)], t_v)
        pltpu.sync_copy(s_hbm.at[pl.ds(e0, C_SUB)], s_v)
        for i in range(C_SUB // 16):
            sl = pl.ds(i * 16, 16)
            tt = t_v[sl]
            pp = p_v[sl]
            d = tt - pp
            t2_v[sl] = tt * tt
            r2_v[sl] = d * d
        c0 = pltpu.async_copy(one_v, tbl_c.at[s_v], add=True, sem=sem0)
        c1 = pltpu.async_copy(t_v, tbl_t.at[s_v], add=True, sem=sem1)
        c2 = pltpu.async_copy(t2_v, tbl_t2.at[s_v], add=True, sem=sem2)
        c3 = pltpu.async_copy(r2_v, tbl_r2.at[s_v], add=True, sem=sem3)
        c0.wait()
        c1.wait()
        c2.wait()
        c3.wait()
        return 0

    lax.fori_loop(0, nsub_w, _sub, 0)
    plsc.subcore_barrier()

    out0 = cid * (4 * S_PAD) + base
    for k, tbl in enumerate((tbl_c, tbl_t, tbl_t2, tbl_r2)):
        pltpu.sync_copy(tbl.at[pl.ds(base, SLICE)], z_v)
        pltpu.sync_copy(z_v, out_hbm.at[pl.ds(out0 + k * S_PAD, SLICE)])


@functools.partial(
    pl.kernel,
    out_type=jax.ShapeDtypeStruct((2 * 4 * S_PAD,), jnp.float32),
    mesh=plsc.VectorSubcoreMesh(core_axis_name="c", subcore_axis_name="s",
                                num_cores=NC, num_subcores=NS),
    scratch_types=[
        pltpu.VMEM((C_SUB,), jnp.int32),
        pltpu.VMEM((C_SUB,), jnp.float32),
        pltpu.VMEM((C_SUB,), jnp.float32),
        pltpu.VMEM((C_SUB,), jnp.float32),
        pltpu.VMEM((C_SUB,), jnp.float32),
        pltpu.VMEM((C_SUB,), jnp.float32),
        pltpu.VMEM((SLICE,), jnp.float32),
        pltpu.VMEM_SHARED((S_PAD,), jnp.float32),
        pltpu.VMEM_SHARED((S_PAD,), jnp.float32),
        pltpu.VMEM_SHARED((S_PAD,), jnp.float32),
        pltpu.VMEM_SHARED((S_PAD,), jnp.float32),
        pltpu.SemaphoreType.DMA,
        pltpu.SemaphoreType.DMA,
        pltpu.SemaphoreType.DMA,
        pltpu.SemaphoreType.DMA,
    ],
)
def _sc_accumulate(*args):
    _sc_accumulate_body(*args)


def _finalize_body(x_ref, o_ref):
    c = x_ref[0] + x_ref[4]
    st = x_ref[1] + x_ref[5]
    st2 = x_ref[2] + x_ref[6]
    sr = x_ref[3] + x_ref[7]
    cs = jnp.maximum(c, 1.0)
    ss_tot = st2 - st * st / cs
    mse = sr / cs
    ss_tot_safe = jnp.where(ss_tot > 1e-8, ss_tot, 1.0)
    r2 = 1.0 - sr / ss_tot_safe
    r2 = jnp.clip(r2, -1.0, 1.0)
    loss_r2 = 1.0 - r2
    use_mse = (c < 5.0) | (ss_tot <= 1e-8)
    sl = jnp.where(use_mse, mse, loss_r2)
    nonempty = c > 0.0
    sl = jnp.where(nonempty, sl, 0.0)
    n_uniq = jnp.sum(nonempty.astype(jnp.float32))
    val = jnp.sum(sl) / jnp.maximum(n_uniq, 1.0)
    o_ref[...] = val[None, None]


def kernel(predictions, targets, station_ids):
    s1 = station_ids.astype(jnp.int32)
    partials = _sc_accumulate(predictions, targets, s1)
    x = partials.reshape(8, S_PAD // 128, 128)
    out = pl.pallas_call(
        _finalize_body,
        out_shape=jax.ShapeDtypeStruct((1, 1), jnp.float32),
    )(x)
    return out.reshape(())


# speedup vs baseline: 79.1250x; 1.0198x over previous
import functools

import jax
import jax.numpy as jnp
from jax import lax
from jax.experimental import pallas as pl
from jax.experimental.pallas import tpu as pltpu
from jax.experimental.pallas import tpu_sc as plsc

NSTATIONS = 50000
S_PAD = 50176
DUMP = S_PAD - 128
NC = 2
NS = 16
NW = NC * NS
SLICE = S_PAD // NS
C_SUB = 1280
NV = C_SUB // 16
PADL = 8
CAPC = 1440


def _sc_accumulate_body(p_hbm, t_hbm, s_hbm, out_hbm,
                        s_v, t_v, p_v, ids_c, e_c, e_t, e_t2, e_r2,
                        d_c, d_t, d_t2, d_r2, cnt_v, z_v,
                        tbl_c, tbl_t, tbl_t2, tbl_r2,
                        sem0, sem1, sem2, sem3):
    cid = lax.axis_index("c")
    sid = lax.axis_index("s")
    wid = cid * NS + sid

    zeros16 = jnp.zeros((16,), jnp.float32)
    fiota = lax.iota(jnp.int32, 16).astype(jnp.float32)
    idx15 = jnp.full((16,), 15, jnp.int32)

    def _zfill(i, _):
        z_v[pl.ds(i * 16, 16)] = zeros16
        return 0
    lax.fori_loop(0, SLICE // 16, _zfill, 0)

    def _efill(i, _):
        e_c[pl.ds(i * 16, 16)] = zeros16
        e_t[pl.ds(i * 16, 16)] = zeros16
        e_t2[pl.ds(i * 16, 16)] = zeros16
        e_r2[pl.ds(i * 16, 16)] = zeros16
        return 0
    lax.fori_loop(0, CAPC // 16, _efill, 0)

    base = sid * SLICE
    pltpu.sync_copy(z_v, tbl_c.at[pl.ds(base, SLICE)])
    pltpu.sync_copy(z_v, tbl_t.at[pl.ds(base, SLICE)])
    pltpu.sync_copy(z_v, tbl_t2.at[pl.ds(base, SLICE)])
    pltpu.sync_copy(z_v, tbl_r2.at[pl.ds(base, SLICE)])
    plsc.subcore_barrier()

    n = p_hbm.shape[0]
    nsub = n // C_SUB
    nsub_w = (nsub - wid + NW - 1) // NW

    def _sub(it, _):
        e0 = (wid + it * NW) * C_SUB
        pltpu.sync_copy(p_hbm.at[pl.ds(e0, C_SUB)], p_v)
        pltpu.sync_copy(t_hbm.at[pl.ds(e0, C_SUB)], t_v)
        pltpu.sync_copy(s_hbm.at[pl.ds(e0, C_SUB)], s_v.at[pl.ds(0, C_SUB)])
        s_v[pl.ds(C_SUB, 16)] = jnp.full((16,), -1, jnp.int32)
        e_c[pl.ds(0, 16)] = zeros16
        e_t[pl.ds(0, 16)] = zeros16
        e_t2[pl.ds(0, 16)] = zeros16
        e_r2[pl.ds(0, 16)] = zeros16

        cur = jnp.int32(PADL)
        carry_t = zeros16
        carry_t2 = zeros16
        carry_r2 = zeros16
        for i in range(NV):
            b = i * 16
            scur = s_v[pl.ds(b, 16)]
            snext = s_v[pl.ds(b + 1, 16)]
            m = scur != snext
            tt = t_v[pl.ds(b, 16)]
            pp = p_v[pl.ds(b, 16)]
            d = tt - pp
            t2x = tt * tt
            r2x = d * d
            cs_t = plsc.cumsum(tt) + carry_t
            cs_t2 = plsc.cumsum(t2x) + carry_t2
            cs_r2 = plsc.cumsum(r2x) + carry_r2
            carry_t = cs_t[idx15]
            carry_t2 = cs_t2[idx15]
            carry_r2 = cs_r2[idx15]
            cs_c = fiota + jnp.float32(b + 1)
            win = pl.ds(cur, 16)
            plsc.store_compressed(ids_c.at[win], scur, mask=m)
            plsc.store_compressed(e_c.at[win], cs_c, mask=m)
            plsc.store_compressed(e_t.at[win], cs_t, mask=m)
            plsc.store_compressed(e_t2.at[win], cs_t2, mask=m)
            plsc.store_compressed(e_r2.at[win], cs_r2, mask=m)
            pc = plsc.all_reduce_population_count(m)
            cur = cur + pc[0]

        dump16 = jnp.full((16,), DUMP, jnp.int32)
        for r in range(8):
            ids_c[pl.ds(cur + r * 16, 16)] = dump16

        k = cur - PADL
        nstr = (k + 127) // 128

        def _scat(j, _):
            for l in range(8):
                bb = PADL + j * 128 + l * 16
                w = pl.ds(bb, 16)
                wp = pl.ds(bb - 1, 16)
                d_c[w] = e_c[w] - e_c[wp]
                d_t[w] = e_t[w] - e_t[wp]
                d_t2[w] = e_t2[w] - e_t2[wp]
                d_r2[w] = e_r2[w] - e_r2[wp]
            rs = pl.ds(PADL + j * 128, 128)
            idx = ids_c.at[rs]
            c0 = pltpu.async_copy(d_c.at[rs], tbl_c.at[idx], add=True, sem=sem0)
            c1 = pltpu.async_copy(d_t.at[rs], tbl_t.at[idx], add=True, sem=sem1)
            c2 = pltpu.async_copy(d_t2.at[rs], tbl_t2.at[idx], add=True, sem=sem2)
            c3 = pltpu.async_copy(d_r2.at[rs], tbl_r2.at[idx], add=True, sem=sem3)
            c0.wait()
            c1.wait()
            c2.wait()
            c3.wait()
            return 0

        lax.fori_loop(0, nstr, _scat, 0)
        return 0

    lax.fori_loop(0, nsub_w, _sub, 0)
    plsc.subcore_barrier()

    out0 = cid * (4 * S_PAD) + base
    for kk, tbl in enumerate((tbl_c, tbl_t, tbl_t2, tbl_r2)):
        pltpu.sync_copy(tbl.at[pl.ds(base, SLICE)], z_v)
        pltpu.sync_copy(z_v, out_hbm.at[pl.ds(out0 + kk * S_PAD, SLICE)])


@functools.partial(
    pl.kernel,
    out_type=jax.ShapeDtypeStruct((2 * 4 * S_PAD,), jnp.float32),
    mesh=plsc.VectorSubcoreMesh(core_axis_name="c", subcore_axis_name="s",
                                num_cores=NC, num_subcores=NS),
    compiler_params=pltpu.CompilerParams(needs_layout_passes=False),
    scratch_types=[
        pltpu.VMEM((C_SUB + 16,), jnp.int32),
        pltpu.VMEM((C_SUB,), jnp.float32),
        pltpu.VMEM((C_SUB,), jnp.float32),
        pltpu.VMEM((CAPC,), jnp.int32),
        pltpu.VMEM((CAPC,), jnp.float32),
        pltpu.VMEM((CAPC,), jnp.float32),
        pltpu.VMEM((CAPC,), jnp.float32),
        pltpu.VMEM((CAPC,), jnp.float32),
        pltpu.VMEM((CAPC,), jnp.float32),
        pltpu.VMEM((CAPC,), jnp.float32),
        pltpu.VMEM((CAPC,), jnp.float32),
        pltpu.VMEM((CAPC,), jnp.float32),
        pltpu.VMEM((16,), jnp.int32),
        pltpu.VMEM((SLICE,), jnp.float32),
        pltpu.VMEM_SHARED((S_PAD,), jnp.float32),
        pltpu.VMEM_SHARED((S_PAD,), jnp.float32),
        pltpu.VMEM_SHARED((S_PAD,), jnp.float32),
        pltpu.VMEM_SHARED((S_PAD,), jnp.float32),
        pltpu.SemaphoreType.DMA,
        pltpu.SemaphoreType.DMA,
        pltpu.SemaphoreType.DMA,
        pltpu.SemaphoreType.DMA,
    ],
)
def _sc_accumulate(*args):
    _sc_accumulate_body(*args)


def _finalize_body(x_ref, o_ref):
    c = x_ref[0] + x_ref[4]
    st = x_ref[1] + x_ref[5]
    st2 = x_ref[2] + x_ref[6]
    sr = x_ref[3] + x_ref[7]
    rows = S_PAD // 128
    gidx = (lax.broadcasted_iota(jnp.int32, (rows, 128), 0) * 128
            + lax.broadcasted_iota(jnp.int32, (rows, 128), 1))
    valid = gidx < NSTATIONS
    cs = jnp.maximum(c, 1.0)
    ss_tot = st2 - st * st / cs
    mse = sr / cs
    ss_tot_safe = jnp.where(ss_tot > 1e-8, ss_tot, 1.0)
    r2 = 1.0 - sr / ss_tot_safe
    r2 = jnp.clip(r2, -1.0, 1.0)
    loss_r2 = 1.0 - r2
    use_mse = (c < 5.0) | (ss_tot <= 1e-8)
    sl = jnp.where(use_mse, mse, loss_r2)
    keep = valid & (c > 0.0)
    sl = jnp.where(keep, sl, 0.0)
    n_uniq = jnp.sum(keep.astype(jnp.float32))
    val = jnp.sum(sl) / jnp.maximum(n_uniq, 1.0)
    o_ref[...] = val[None, None]


def kernel(predictions, targets, station_ids):
    s1 = station_ids.astype(jnp.int32)
    partials = _sc_accumulate(predictions, targets, s1)
    x = partials.reshape(8, S_PAD // 128, 128)
    out = pl.pallas_call(
        _finalize_body,
        out_shape=jax.ShapeDtypeStruct((1, 1), jnp.float32),
    )(x)
    return out.reshape(())
